# baseline (speedup 1.0000x reference)
import functools

import jax
import jax.numpy as jnp
from jax import lax
from jax.experimental import pallas as pl
from jax.experimental.pallas import tpu as pltpu
from jax.experimental.pallas import tpu_sc as plsc

SCALE_CONST = 4
NC, NS, L = 2, 16, 16
NW = NC * NS


def _build(batch, chans, h, w, crows, interpret=False):
    n = h * w
    chunk = crows * w
    oh, ow = h * SCALE_CONST, w * SCALE_CONST
    rb = oh // NW
    vpr = w // L
    nchunk = n // chunk
    npair = batch * nchunk
    ppt = npair // NS
    assert n % chunk == 0 and oh % NW == 0 and w % L == 0
    assert npair % NS == 0
    jshift = nchunk.bit_length() - 1
    assert (1 << jshift) == nchunk

    def body(img_hbm, flow_hbm, scl_hbm, out_hbm,
             acc, fyb, fxb, cbufs, sclb, minb, maxb, mint, maxt,
             shmin, shmax, sem, osem):
        cid = lax.axis_index("c")
        sid = lax.axis_index("s")
        wid = sid * NC + cid
        lo = wid * rb

        def any_true(mvec):
            return plsc.all_reduce_population_count(mvec)[0] > 0

        pltpu.sync_copy(scl_hbm, sclb)
        scale_v = sclb[...]

        iota_f = lax.iota(jnp.int32, L).astype(jnp.float32)
        lo_f = lo.astype(jnp.float32)
        band_lo = jnp.full((L,), lo_f - 1.0, jnp.float32)
        band_hi = jnp.full((L,), lo_f + rb, jnp.float32)
        zeros = jnp.zeros((L,), jnp.float32)
        scope = jax.named_scope

        def range_pair(i, c):
            k = sid * ppt + i
            b = k >> jshift
            j = k - (b << jshift)
            pltpu.sync_copy(
                flow_hbm.at[b, 1, pl.ds(j * crows, crows), :], fyb)

            def mm_row(r, carry):
                row_f = (j * crows + r).astype(jnp.float32)
                gy = jnp.full((L,), row_f, jnp.float32)

                def mm(v, rc):
                    mn, mx = rc
                    y = (gy + fyb[r, pl.ds(v * L, L)]) * scale_v
                    return jnp.minimum(mn, y), jnp.maximum(mx, y)
                return lax.fori_loop(0, vpr, mm, carry)
            mn, mx = lax.fori_loop(
                0, crows, mm_row,
                (jnp.full((L,), 3.0e38, jnp.float32),
                 jnp.full((L,), -3.0e38, jnp.float32)))
            minb[pl.ds(i * L, L)] = mn
            maxb[pl.ds(i * L, L)] = mx
            return c
        with scope("phase0_range"):
            lax.fori_loop(0, ppt, range_pair, 0)
            pltpu.sync_copy(minb, shmin.at[pl.ds(sid * ppt * L, ppt * L)])
            pltpu.sync_copy(maxb, shmax.at[pl.ds(sid * ppt * L, ppt * L)])
            plsc.subcore_barrier()
            pltpu.sync_copy(shmin, mint)
            pltpu.sync_copy(shmax, maxt)

        def zero_rows(r0, nrows):
            def zloop(i, c):
                for k in range(ow // (8 * L)):
                    off = k * (8 * L)
                    for kk in range(8):
                        acc[r0 + i, pl.ds(off + kk * L, L)] = zeros
                return c
            lax.fori_loop(0, nrows, zloop, 0)

        with scope("zero_init"):
            zero_rows(0, chans * rb)

        def per_batch(b, carry):
            def chunk_loop(j, c):
                k16 = (b * nchunk + j) * L
                cmin = mint[pl.ds(k16, L)]
                cmax = maxt[pl.ds(k16, L)]
                hit = (cmax >= band_lo) & (cmin < band_hi)

                @pl.when(any_true(hit))
                def _process():
                    rsl = pl.ds(j * crows, crows)
                    cps = [
                        pltpu.async_copy(
                            flow_hbm.at[b, 1, rsl, :], fyb, sem),
                        pltpu.async_copy(
                            flow_hbm.at[b, 0, rsl, :], fxb, sem),
                    ] + [
                        pltpu.async_copy(
                            img_hbm.at[b, ch, rsl, :], cbufs[ch], sem)
                        for ch in range(chans)
                    ]
                    with scope("fetch_wait"):
                        for cp in cps:
                            cp.wait()

                    def row_loop(r, cr):
                        row_f = (j * crows + r).astype(jnp.float32)
                        gy = jnp.full((L,), row_f, jnp.float32)

                        def proc(v):
                            sl = pl.ds(v * L, L)
                            y = (gy + fyb[r, sl]) * scale_v
                            gx = iota_f + (v * L).astype(jnp.float32)
                            x = (gx + fxb[r, sl]) * scale_v
                            xt = x.astype(jnp.int32)
                            x0 = xt - jnp.where(
                                xt.astype(jnp.float32) > x, 1, 0)
                            yt = y.astype(jnp.int32)
                            y0 = yt - jnp.where(
                                yt.astype(jnp.float32) > y, 1, 0)
                            wx1 = x - x0.astype(jnp.float32)
                            wx0 = 1.0 - wx1
                            wy1 = y - y0.astype(jnp.float32)
                            wy0 = 1.0 - wy1
                            vals = [cb[r, sl] for cb in cbufs]
                            ly = y0 - lo
                            x1 = x0 + 1
                            for lyv, wy in ((ly, wy0), (ly + 1, wy1)):
                                my = (lyv >= 0) & (lyv < rb)
                                rowi = jnp.where(my, lyv, 0)
                                for xv, wx in ((x0, wx0), (x1, wx1)):
                                    m = my & (xv >= 0) & (xv < ow)
                                    wgt = wy * wx
                                    coli = jnp.where(m, xv, 0)
                                    for ch in range(chans):
                                        plsc.addupdate_scatter(
                                            acc,
                                            [rowi + ch * rb, coli],
                                            vals[ch] * wgt, mask=m)
                        plsc.parallel_loop(0, vpr, unroll=4)(proc)
                        return cr
                    with scope("proc"):
                        lax.fori_loop(0, crows, row_loop, 0)
                return c
            with scope("chunks"):
                lax.fori_loop(0, nchunk, chunk_loop, 0)

            flushes = [
                pltpu.async_copy(
                    acc.at[pl.ds(ch * rb, rb), :],
                    out_hbm.at[b, ch, pl.ds(lo, rb), :],
                    osem)
                for ch in range(chans)
            ]
            last = b == batch - 1
            with scope("flush_zero"):
                for ch in range(chans):
                    flushes[ch].wait()

                    @pl.when(jnp.logical_not(last))
                    def _rezero():
                        zero_rows(ch * rb, rb)
            return carry

        lax.fori_loop(0, batch, per_batch, 0)

    mesh = plsc.VectorSubcoreMesh(core_axis_name="c", subcore_axis_name="s",
                                  num_cores=NC, num_subcores=NS)

    def wrapped(img_hbm, flow_hbm, scl_hbm, out_hbm,
                acc, fyb, fxb, c0, c1, c2, sclb,
                minb, maxb, mint, maxt, shmin, shmax, sem, osem):
        return body(img_hbm, flow_hbm, scl_hbm, out_hbm,
                    acc, fyb, fxb, [c0, c1, c2], sclb,
                    minb, maxb, mint, maxt, shmin, shmax, sem, osem)

    @jax.jit
    def warp(img, flow, scl):
        return pl.kernel(
            wrapped,
            out_type=jax.ShapeDtypeStruct((batch, chans, oh, ow),
                                          jnp.float32),
            mesh=mesh,
            interpret=interpret,
            compiler_params=pltpu.CompilerParams(needs_layout_passes=False),
            scratch_types=[
                pltpu.VMEM((chans * rb, ow), jnp.float32),
                pltpu.VMEM((crows, w), jnp.float32),
                pltpu.VMEM((crows, w), jnp.float32),
                pltpu.VMEM((crows, w), jnp.float32),
                pltpu.VMEM((crows, w), jnp.float32),
                pltpu.VMEM((crows, w), jnp.float32),
                pltpu.VMEM((L,), jnp.float32),
                pltpu.VMEM((ppt * L,), jnp.float32),
                pltpu.VMEM((ppt * L,), jnp.float32),
                pltpu.VMEM((npair * L,), jnp.float32),
                pltpu.VMEM((npair * L,), jnp.float32),
                pltpu.VMEM_SHARED((npair * L,), jnp.float32),
                pltpu.VMEM_SHARED((npair * L,), jnp.float32),
                pltpu.SemaphoreType.DMA,
                pltpu.SemaphoreType.DMA,
            ],
        )(img, flow, scl)

    def run(img, flow, scale):
        scl = jnp.full((L,), scale, jnp.float32)
        return warp(img, flow, scl)

    return run


_run = _build(8, 3, 256, 256, 8)


def kernel(img, flow, scale):
    return _run(img, flow, scale)


# speedup vs baseline: 1.0384x; 1.0384x over previous
import functools

import jax
import jax.numpy as jnp
from jax import lax
from jax.experimental import pallas as pl
from jax.experimental.pallas import tpu as pltpu
from jax.experimental.pallas import tpu_sc as plsc

SCALE_CONST = 4
NC, NS, L = 2, 16, 16
NW = NC * NS


def _build(batch, chans, h, w, crows, interpret=False):
    n = h * w
    chunk = crows * w
    oh, ow = h * SCALE_CONST, w * SCALE_CONST
    rb = oh // NW
    vpr = w // L
    nchunk = n // chunk
    npair = batch * nchunk
    ppt = npair // NS
    assert n % chunk == 0 and oh % NW == 0 and w % L == 0
    assert npair % NS == 0
    jshift = nchunk.bit_length() - 1
    assert (1 << jshift) == nchunk

    def body(img_hbm, flow_hbm, scl_hbm, out_hbm,
             acc, fyb, fxb, cbufs, sclb, minb, maxb, mint, maxt,
             shmin, shmax, sem, osem):
        cid = lax.axis_index("c")
        sid = lax.axis_index("s")
        wid = sid * NC + cid
        lo = wid * rb

        def any_true(mvec):
            return plsc.all_reduce_population_count(mvec)[0] > 0

        pltpu.sync_copy(scl_hbm, sclb)
        scale_v = sclb[...]

        iota_f = lax.iota(jnp.int32, L).astype(jnp.float32)
        lo_f = lo.astype(jnp.float32)
        band_lo = jnp.full((L,), lo_f - 1.0, jnp.float32)
        band_hi = jnp.full((L,), lo_f + rb, jnp.float32)
        zeros = jnp.zeros((L,), jnp.float32)
        scope = jax.named_scope

        def range_pair(i, c):
            k = sid * ppt + i
            b = k >> jshift
            j = k - (b << jshift)
            pltpu.sync_copy(
                flow_hbm.at[b, 1, pl.ds(j * crows, crows), :], fyb)

            def mm_row(r, carry):
                row_f = (j * crows + r).astype(jnp.float32)
                gy = jnp.full((L,), row_f, jnp.float32)

                def mm(v, rc):
                    mn, mx = rc
                    y = (gy + fyb[r, pl.ds(v * L, L)]) * scale_v
                    return jnp.minimum(mn, y), jnp.maximum(mx, y)
                return lax.fori_loop(0, vpr, mm, carry)
            mn, mx = lax.fori_loop(
                0, crows, mm_row,
                (jnp.full((L,), 3.0e38, jnp.float32),
                 jnp.full((L,), -3.0e38, jnp.float32)))
            minb[pl.ds(i * L, L)] = mn
            maxb[pl.ds(i * L, L)] = mx
            return c
        with scope("phase0_range"):
            lax.fori_loop(0, ppt, range_pair, 0)
            pltpu.sync_copy(minb, shmin.at[pl.ds(sid * ppt * L, ppt * L)])
            pltpu.sync_copy(maxb, shmax.at[pl.ds(sid * ppt * L, ppt * L)])
            plsc.subcore_barrier()
            pltpu.sync_copy(shmin, mint)
            pltpu.sync_copy(shmax, maxt)

        def zero_rows(r0, nrows):
            def zloop(i, c):
                for k in range(ow // (8 * L)):
                    off = k * (8 * L)
                    for kk in range(8):
                        acc[r0 + i, pl.ds(off + kk * L, L)] = zeros
                return c
            lax.fori_loop(0, nrows, zloop, 0)

        with scope("zero_init"):
            zero_rows(0, chans * rb)

        def per_batch(b, carry):
            def chunk_loop(j, c):
                k16 = (b * nchunk + j) * L
                cmin = mint[pl.ds(k16, L)]
                cmax = maxt[pl.ds(k16, L)]
                hit = (cmax >= band_lo) & (cmin < band_hi)

                @pl.when(any_true(hit))
                def _process():
                    rsl = pl.ds(j * crows, crows)
                    cps = [
                        pltpu.async_copy(
                            flow_hbm.at[b, 1, rsl, :], fyb, sem),
                        pltpu.async_copy(
                            flow_hbm.at[b, 0, rsl, :], fxb, sem),
                    ] + [
                        pltpu.async_copy(
                            img_hbm.at[b, ch, rsl, :], cbufs[ch], sem)
                        for ch in range(chans)
                    ]
                    with scope("fetch_wait"):
                        for cp in cps:
                            cp.wait()

                    def row_loop(r, cr):
                        row_f = (j * crows + r).astype(jnp.float32)
                        gy = jnp.full((L,), row_f, jnp.float32)

                        def proc(v):
                            sl = pl.ds(v * L, L)
                            y = (gy + fyb[r, sl]) * scale_v
                            gx = iota_f + (v * L).astype(jnp.float32)
                            x = (gx + fxb[r, sl]) * scale_v
                            xt = x.astype(jnp.int32)
                            x0 = xt - jnp.where(
                                xt.astype(jnp.float32) > x, 1, 0)
                            yt = y.astype(jnp.int32)
                            y0 = yt - jnp.where(
                                yt.astype(jnp.float32) > y, 1, 0)
                            wx1 = x - x0.astype(jnp.float32)
                            wx0 = 1.0 - wx1
                            wy1 = y - y0.astype(jnp.float32)
                            wy0 = 1.0 - wy1
                            vals = [cb[r, sl] for cb in cbufs]
                            ly = y0 - lo
                            x1 = x0 + 1
                            for lyv, wy in ((ly, wy0), (ly + 1, wy1)):
                                my = (lyv >= 0) & (lyv < rb)
                                rowi = jnp.where(my, lyv, 0)
                                for xv, wx in ((x0, wx0), (x1, wx1)):
                                    m = my & (xv >= 0) & (xv < ow)
                                    wgt = wy * wx
                                    coli = jnp.where(m, xv, 0)
                                    for ch in range(chans):
                                        plsc.addupdate_scatter(
                                            acc,
                                            [rowi + ch * rb, coli],
                                            vals[ch] * wgt, mask=m)
                        plsc.parallel_loop(0, vpr, unroll=2)(proc)
                        return cr
                    with scope("proc"):
                        lax.fori_loop(0, crows, row_loop, 0)
                return c
            with scope("chunks"):
                lax.fori_loop(0, nchunk, chunk_loop, 0)

            flushes = [
                pltpu.async_copy(
                    acc.at[pl.ds(ch * rb, rb), :],
                    out_hbm.at[b, ch, pl.ds(lo, rb), :],
                    osem)
                for ch in range(chans)
            ]
            last = b == batch - 1
            with scope("flush_zero"):
                for ch in range(chans):
                    flushes[ch].wait()

                    @pl.when(jnp.logical_not(last))
                    def _rezero():
                        zero_rows(ch * rb, rb)
            return carry

        lax.fori_loop(0, batch, per_batch, 0)

    mesh = plsc.VectorSubcoreMesh(core_axis_name="c", subcore_axis_name="s",
                                  num_cores=NC, num_subcores=NS)

    def wrapped(img_hbm, flow_hbm, scl_hbm, out_hbm,
                acc, fyb, fxb, c0, c1, c2, sclb,
                minb, maxb, mint, maxt, shmin, shmax, sem, osem):
        return body(img_hbm, flow_hbm, scl_hbm, out_hbm,
                    acc, fyb, fxb, [c0, c1, c2], sclb,
                    minb, maxb, mint, maxt, shmin, shmax, sem, osem)

    @jax.jit
    def warp(img, flow, scl):
        return pl.kernel(
            wrapped,
            out_type=jax.ShapeDtypeStruct((batch, chans, oh, ow),
                                          jnp.float32),
            mesh=mesh,
            interpret=interpret,
            compiler_params=pltpu.CompilerParams(needs_layout_passes=False),
            scratch_types=[
                pltpu.VMEM((chans * rb, ow), jnp.float32),
                pltpu.VMEM((crows, w), jnp.float32),
                pltpu.VMEM((crows, w), jnp.float32),
                pltpu.VMEM((crows, w), jnp.float32),
                pltpu.VMEM((crows, w), jnp.float32),
                pltpu.VMEM((crows, w), jnp.float32),
                pltpu.VMEM((L,), jnp.float32),
                pltpu.VMEM((ppt * L,), jnp.float32),
                pltpu.VMEM((ppt * L,), jnp.float32),
                pltpu.VMEM((npair * L,), jnp.float32),
                pltpu.VMEM((npair * L,), jnp.float32),
                pltpu.VMEM_SHARED((npair * L,), jnp.float32),
                pltpu.VMEM_SHARED((npair * L,), jnp.float32),
                pltpu.SemaphoreType.DMA,
                pltpu.SemaphoreType.DMA,
            ],
        )(img, flow, scl)

    def run(img, flow, scale):
        scl = jnp.full((L,), scale, jnp.float32)
        return warp(img, flow, scl)

    return run


_run = _build(8, 3, 256, 256, 8)


def kernel(img, flow, scale):
    return _run(img, flow, scale)


# speedup vs baseline: 1.0603x; 1.0211x over previous
import functools

import jax
import jax.numpy as jnp
from jax import lax
from jax.experimental import pallas as pl
from jax.experimental.pallas import tpu as pltpu
from jax.experimental.pallas import tpu_sc as plsc

SCALE_CONST = 4
NC, NS, L = 2, 16, 16
NW = NC * NS


def _build(batch, chans, h, w, crows, interpret=False):
    n = h * w
    chunk = crows * w
    oh, ow = h * SCALE_CONST, w * SCALE_CONST
    rb = oh // NW
    vpr = w // L
    nchunk = n // chunk
    npair = batch * nchunk
    ppt = npair // NS
    assert n % chunk == 0 and oh % NW == 0 and w % L == 0
    assert npair % NS == 0
    jshift = nchunk.bit_length() - 1
    assert (1 << jshift) == nchunk

    def body(img_hbm, flow_hbm, scl_hbm, out_hbm,
             acc, fyb, fxb, cbufs, sclb, minb, maxb, mint, maxt,
             shmin, shmax, isem, osem):
        cid = lax.axis_index("c")
        sid = lax.axis_index("s")
        wid = sid * NC + cid
        lo = wid * rb

        def any_true(mvec):
            return plsc.all_reduce_population_count(mvec)[0] > 0

        pltpu.sync_copy(scl_hbm, sclb)
        scale_v = sclb[...]

        iota_f = lax.iota(jnp.int32, L).astype(jnp.float32)
        lo_f = lo.astype(jnp.float32)
        band_lo = jnp.full((L,), lo_f - 1.0, jnp.float32)
        band_hi = jnp.full((L,), lo_f + rb, jnp.float32)
        zeros = jnp.zeros((L,), jnp.float32)
        scope = jax.named_scope

        def fy_src(b, j):
            return flow_hbm.at[b, 1, pl.ds(j * crows, crows), :]

        def p0_src(i):
            k = sid * ppt + i
            b = k >> jshift
            return fy_src(b, k - (b << jshift))

        def p0_minmax(i, par):
            k = sid * ppt + i
            b = k >> jshift
            jj = k - (b << jshift)
            pltpu.make_async_copy(p0_src(i), fyb.at[par], isem.at[par]).wait()

            def mm_row(r, carry):
                row_f = (jj * crows + r).astype(jnp.float32)
                gy = jnp.full((L,), row_f, jnp.float32)

                def mm(v, rc):
                    mn, mx = rc
                    y = (gy + fyb[par, r, pl.ds(v * L, L)]) * scale_v
                    return jnp.minimum(mn, y), jnp.maximum(mx, y)
                return lax.fori_loop(0, vpr, mm, carry)
            mn, mx = lax.fori_loop(
                0, crows, mm_row,
                (jnp.full((L,), 3.0e38, jnp.float32),
                 jnp.full((L,), -3.0e38, jnp.float32)))
            minb[pl.ds(i * L, L)] = mn
            maxb[pl.ds(i * L, L)] = mx

        with scope("phase0_range"):
            pltpu.async_copy(p0_src(0), fyb.at[0], isem.at[0])
            for i in range(ppt):
                if i + 1 < ppt:
                    pltpu.async_copy(p0_src(i + 1), fyb.at[(i + 1) & 1],
                                     isem.at[(i + 1) & 1])
                p0_minmax(i, i & 1)
            pltpu.sync_copy(minb, shmin.at[pl.ds(sid * ppt * L, ppt * L)])
            pltpu.sync_copy(maxb, shmax.at[pl.ds(sid * ppt * L, ppt * L)])
            plsc.subcore_barrier()
            pltpu.sync_copy(shmin, mint)
            pltpu.sync_copy(shmax, maxt)

        def zero_rows(r0, nrows):
            def zloop(i, c):
                for k in range(ow // (8 * L)):
                    off = k * (8 * L)
                    for kk in range(8):
                        acc[r0 + i, pl.ds(off + kk * L, L)] = zeros
                return c
            lax.fori_loop(0, nrows, zloop, 0)

        with scope("zero_init"):
            zero_rows(0, chans * rb)

        def chunk_srcs(b, j):
            rsl = pl.ds(j * crows, crows)
            return ([flow_hbm.at[b, 1, rsl, :], flow_hbm.at[b, 0, rsl, :]]
                    + [img_hbm.at[b, ch, rsl, :] for ch in range(chans)])

        def fire(b, j, par):
            for src, dst in zip(chunk_srcs(b, j), [fyb, fxb] + cbufs):
                pltpu.async_copy(src, dst.at[par], isem.at[par])

        def drain(b, j, par):
            for src, dst in zip(chunk_srcs(b, j), [fyb, fxb] + cbufs):
                pltpu.make_async_copy(src, dst.at[par], isem.at[par]).wait()

        def do_proc(b, j, par):
            with scope("fetch_wait"):
                drain(b, j, par)

            def row_loop(r, cr):
                row_f = (j * crows + r).astype(jnp.float32)
                gy = jnp.full((L,), row_f, jnp.float32)

                def proc(v):
                    sl = pl.ds(v * L, L)
                    y = (gy + fyb[par, r, sl]) * scale_v
                    gx = iota_f + (v * L).astype(jnp.float32)
                    x = (gx + fxb[par, r, sl]) * scale_v
                    xt = x.astype(jnp.int32)
                    x0 = xt - jnp.where(xt.astype(jnp.float32) > x, 1, 0)
                    yt = y.astype(jnp.int32)
                    y0 = yt - jnp.where(yt.astype(jnp.float32) > y, 1, 0)
                    wx1 = x - x0.astype(jnp.float32)
                    wx0 = 1.0 - wx1
                    wy1 = y - y0.astype(jnp.float32)
                    wy0 = 1.0 - wy1
                    vals = [cb[par, r, sl] for cb in cbufs]
                    ly = y0 - lo
                    x1 = x0 + 1
                    for lyv, wy in ((ly, wy0), (ly + 1, wy1)):
                        my = (lyv >= 0) & (lyv < rb)
                        rowi = jnp.where(my, lyv, 0)
                        for xv, wx in ((x0, wx0), (x1, wx1)):
                            m = my & (xv >= 0) & (xv < ow)
                            wgt = wy * wx
                            coli = jnp.where(m, xv, 0)
                            for ch in range(chans):
                                plsc.addupdate_scatter(
                                    acc, [rowi + ch * rb, coli],
                                    vals[ch] * wgt, mask=m)
                plsc.parallel_loop(0, vpr, unroll=2)(proc)
                return cr
            with scope("proc"):
                lax.fori_loop(0, crows, row_loop, 0)

        def flush_srcs(b):
            return [(acc.at[pl.ds(ch * rb, rb), :],
                     out_hbm.at[b, ch, pl.ds(lo, rb), :])
                    for ch in range(chans)]

        def per_batch(b, carry):
            @pl.when(b > 0)
            def _drain_flush():
                with scope("flush_zero"):
                    for src, dst in flush_srcs(b - 1):
                        pltpu.make_async_copy(src, dst, osem).wait()
                    zero_rows(0, chans * rb)

            def chunk_loop(j, c):
                pend, buf = c
                k16 = (b * nchunk + j) * L
                cmin = mint[pl.ds(k16, L)]
                cmax = maxt[pl.ds(k16, L)]
                hit = any_true((cmax >= band_lo) & (cmin < band_hi))

                @pl.when(hit)
                def _advance():
                    fire(b, j, buf)

                    @pl.when(pend >= 0)
                    def _proc_pending():
                        do_proc(b, pend, 1 - buf)
                return (jnp.where(hit, j, pend),
                        jnp.where(hit, 1 - buf, buf))
            pend, buf = lax.fori_loop(
                0, nchunk, chunk_loop,
                (jnp.int32(-1), jnp.int32(0)))

            @pl.when(pend >= 0)
            def _proc_tail():
                do_proc(b, pend, 1 - buf)

            for src, dst in flush_srcs(b):
                pltpu.async_copy(src, dst, osem)
            return carry

        lax.fori_loop(0, batch, per_batch, 0)

        with scope("final_drain"):
            for src, dst in flush_srcs(batch - 1):
                pltpu.make_async_copy(src, dst, osem).wait()

    mesh = plsc.VectorSubcoreMesh(core_axis_name="c", subcore_axis_name="s",
                                  num_cores=NC, num_subcores=NS)

    def wrapped(img_hbm, flow_hbm, scl_hbm, out_hbm,
                acc, fyb, fxb, c0, c1, c2, sclb,
                minb, maxb, mint, maxt, shmin, shmax, isem, osem):
        return body(img_hbm, flow_hbm, scl_hbm, out_hbm,
                    acc, fyb, fxb, [c0, c1, c2], sclb,
                    minb, maxb, mint, maxt, shmin, shmax, isem, osem)

    @jax.jit
    def warp(img, flow, scl):
        return pl.kernel(
            wrapped,
            out_type=jax.ShapeDtypeStruct((batch, chans, oh, ow),
                                          jnp.float32),
            mesh=mesh,
            interpret=interpret,
            compiler_params=pltpu.CompilerParams(needs_layout_passes=False),
            scratch_types=[
                pltpu.VMEM((chans * rb, ow), jnp.float32),
                pltpu.VMEM((2, crows, w), jnp.float32),
                pltpu.VMEM((2, crows, w), jnp.float32),
                pltpu.VMEM((2, crows, w), jnp.float32),
                pltpu.VMEM((2, crows, w), jnp.float32),
                pltpu.VMEM((2, crows, w), jnp.float32),
                pltpu.VMEM((L,), jnp.float32),
                pltpu.VMEM((ppt * L,), jnp.float32),
                pltpu.VMEM((ppt * L,), jnp.float32),
                pltpu.VMEM((npair * L,), jnp.float32),
                pltpu.VMEM((npair * L,), jnp.float32),
                pltpu.VMEM_SHARED((npair * L,), jnp.float32),
                pltpu.VMEM_SHARED((npair * L,), jnp.float32),
                pltpu.SemaphoreType.DMA((2,)),
                pltpu.SemaphoreType.DMA,
            ],
        )(img, flow, scl)

    def run(img, flow, scale):
        scl = jnp.full((L,), scale, jnp.float32)
        return warp(img, flow, scl)

    return run


_run = _build(8, 3, 256, 256, 8)


def kernel(img, flow, scale):
    return _run(img, flow, scale)


# speedup vs baseline: 1.1572x; 1.0914x over previous
import functools

import jax
import jax.numpy as jnp
from jax import lax
from jax.experimental import pallas as pl
from jax.experimental.pallas import tpu as pltpu
from jax.experimental.pallas import tpu_sc as plsc

SCALE_CONST = 4
NC, NS, L = 2, 16, 16
NW = NC * NS


def _build(batch, chans, h, w, crows, interpret=False):
    n = h * w
    chunk = crows * w
    oh, ow = h * SCALE_CONST, w * SCALE_CONST
    rb = oh // NW
    vpr = w // L
    nchunk = n // chunk
    npair = batch * nchunk
    ppt = npair // NS
    assert n % chunk == 0 and oh % NW == 0 and w % L == 0
    assert npair % NS == 0
    jshift = nchunk.bit_length() - 1
    assert (1 << jshift) == nchunk

    def body(img_hbm, flow_hbm, scl_hbm, out_hbm,
             acc, fyb, fxb, cbufs, sclb, minb, maxb, mint, maxt,
             shmin, shmax, isem, osem):
        cid = lax.axis_index("c")
        sid = lax.axis_index("s")
        wid = sid * NC + cid
        lo = wid * rb

        def any_true(mvec):
            return plsc.all_reduce_population_count(mvec)[0] > 0

        pltpu.sync_copy(scl_hbm, sclb)
        scale_v = sclb[...]

        iota_f = lax.iota(jnp.int32, L).astype(jnp.float32)
        lo_f = lo.astype(jnp.float32)
        band_lo = jnp.full((L,), lo_f - 1.0, jnp.float32)
        band_hi = jnp.full((L,), lo_f + rb, jnp.float32)
        zeros = jnp.zeros((L,), jnp.float32)
        scope = jax.named_scope

        def fy_src(b, j):
            return flow_hbm.at[b, 1, pl.ds(j * crows, crows), :]

        def p0_src(i):
            k = sid * ppt + i
            b = k >> jshift
            return fy_src(b, k - (b << jshift))

        def p0_minmax(i, par):
            k = sid * ppt + i
            b = k >> jshift
            jj = k - (b << jshift)
            pltpu.make_async_copy(p0_src(i), fyb.at[par], isem.at[par]).wait()

            def mm_row(r, carry):
                row_f = (jj * crows + r).astype(jnp.float32)
                gy = jnp.full((L,), row_f, jnp.float32)

                def mm(v, rc):
                    mn, mx = rc
                    y = (gy + fyb[par, r, pl.ds(v * L, L)]) * scale_v
                    return jnp.minimum(mn, y), jnp.maximum(mx, y)
                return lax.fori_loop(0, vpr, mm, carry)
            mn, mx = lax.fori_loop(
                0, crows, mm_row,
                (jnp.full((L,), 3.0e38, jnp.float32),
                 jnp.full((L,), -3.0e38, jnp.float32)))
            minb[pl.ds(i * L, L)] = mn
            maxb[pl.ds(i * L, L)] = mx

        with scope("phase0_range"):
            pltpu.async_copy(p0_src(0), fyb.at[0], isem.at[0])
            for i in range(ppt):
                if i + 1 < ppt:
                    pltpu.async_copy(p0_src(i + 1), fyb.at[(i + 1) & 1],
                                     isem.at[(i + 1) & 1])
                p0_minmax(i, i & 1)
            pltpu.sync_copy(minb, shmin.at[pl.ds(sid * ppt * L, ppt * L)])
            pltpu.sync_copy(maxb, shmax.at[pl.ds(sid * ppt * L, ppt * L)])
            plsc.subcore_barrier()
            pltpu.sync_copy(shmin, mint)
            pltpu.sync_copy(shmax, maxt)

        def zero_rows(r0, nrows):
            def zloop(i, c):
                for k in range(ow // (8 * L)):
                    off = k * (8 * L)
                    for kk in range(8):
                        acc[r0 + i, pl.ds(off + kk * L, L)] = zeros
                return c
            lax.fori_loop(0, nrows, zloop, 0)

        with scope("zero_init"):
            zero_rows(0, chans * rb)

        def chunk_srcs(b, j):
            rsl = pl.ds(j * crows, crows)
            return ([flow_hbm.at[b, 1, rsl, :], flow_hbm.at[b, 0, rsl, :]]
                    + [img_hbm.at[b, ch, rsl, :] for ch in range(chans)])

        def fire(b, j, par):
            for src, dst in zip(chunk_srcs(b, j), [fyb, fxb] + cbufs):
                pltpu.async_copy(src, dst.at[par], isem.at[par])

        def drain(b, j, par):
            for src, dst in zip(chunk_srcs(b, j), [fyb, fxb] + cbufs):
                pltpu.make_async_copy(src, dst.at[par], isem.at[par]).wait()

        def do_proc(b, j, par):
            with scope("fetch_wait"):
                drain(b, j, par)

            def row_loop(r, cr):
                row_f = (j * crows + r).astype(jnp.float32)
                gy = jnp.full((L,), row_f, jnp.float32)

                def proc(v):
                    sl = pl.ds(v * L, L)
                    y = (gy + fyb[par, r, sl]) * scale_v
                    gx = iota_f + (v * L).astype(jnp.float32)
                    x = (gx + fxb[par, r, sl]) * scale_v
                    xt = x.astype(jnp.int32)
                    x0 = xt - jnp.where(xt.astype(jnp.float32) > x, 1, 0)
                    yt = y.astype(jnp.int32)
                    y0 = yt - jnp.where(yt.astype(jnp.float32) > y, 1, 0)
                    wx1 = x - x0.astype(jnp.float32)
                    wx0 = 1.0 - wx1
                    wy1 = y - y0.astype(jnp.float32)
                    wy0 = 1.0 - wy1
                    vals = [cb[par, r, sl] for cb in cbufs]
                    ly = y0 - lo
                    x1 = x0 + 1
                    for lyv, wy in ((ly, wy0), (ly + 1, wy1)):
                        my = (lyv >= 0) & (lyv < rb)
                        rowi = jnp.where(my, lyv, 0)
                        for xv, wx in ((x0, wx0), (x1, wx1)):
                            m = my & (xv >= 0) & (xv < ow)
                            wgt = wy * wx
                            coli = jnp.where(m, xv, 0)
                            for ch in range(chans):
                                plsc.addupdate_scatter(
                                    acc, [rowi + ch * rb, coli],
                                    vals[ch] * wgt, mask=m)
                plsc.parallel_loop(0, vpr, unroll=2)(proc)
                return cr
            with scope("proc"):
                lax.fori_loop(0, crows, row_loop, 0)

        def flush_srcs(b):
            return [(acc.at[pl.ds(ch * rb, rb), :],
                     out_hbm.at[b, ch, pl.ds(lo, rb), :])
                    for ch in range(chans)]

        def per_batch(b, carry):
            @pl.when(b > 0)
            def _drain_flush():
                with scope("flush_zero"):
                    for ch, (src, dst) in enumerate(flush_srcs(b - 1)):
                        pltpu.make_async_copy(src, dst, osem).wait()
                        zero_rows(ch * rb, rb)

            def chunk_loop(j, c):
                pend, buf = c
                k16 = (b * nchunk + j) * L
                cmin = mint[pl.ds(k16, L)]
                cmax = maxt[pl.ds(k16, L)]
                hit = any_true((cmax >= band_lo) & (cmin < band_hi))

                @pl.when(hit)
                def _advance():
                    fire(b, j, buf)

                    @pl.when(pend >= 0)
                    def _proc_pending():
                        do_proc(b, pend, 1 - buf)
                return (jnp.where(hit, j, pend),
                        jnp.where(hit, 1 - buf, buf))
            pend, buf = lax.fori_loop(
                0, nchunk, chunk_loop,
                (jnp.int32(-1), jnp.int32(0)))

            @pl.when(pend >= 0)
            def _proc_tail():
                do_proc(b, pend, 1 - buf)

            for src, dst in flush_srcs(b):
                pltpu.async_copy(src, dst, osem)
            return carry

        lax.fori_loop(0, batch, per_batch, 0)

        with scope("final_drain"):
            for src, dst in flush_srcs(batch - 1):
                pltpu.make_async_copy(src, dst, osem).wait()

    mesh = plsc.VectorSubcoreMesh(core_axis_name="c", subcore_axis_name="s",
                                  num_cores=NC, num_subcores=NS)

    def wrapped(img_hbm, flow_hbm, scl_hbm, out_hbm,
                acc, fyb, fxb, c0, c1, c2, sclb,
                minb, maxb, mint, maxt, shmin, shmax, isem, osem):
        return body(img_hbm, flow_hbm, scl_hbm, out_hbm,
                    acc, fyb, fxb, [c0, c1, c2], sclb,
                    minb, maxb, mint, maxt, shmin, shmax, isem, osem)

    @jax.jit
    def warp(img, flow, scl):
        return pl.kernel(
            wrapped,
            out_type=jax.ShapeDtypeStruct((batch, chans, oh, ow),
                                          jnp.float32),
            mesh=mesh,
            interpret=interpret,
            compiler_params=pltpu.CompilerParams(needs_layout_passes=False),
            scratch_types=[
                pltpu.VMEM((chans * rb, ow), jnp.float32),
                pltpu.VMEM((2, crows, w), jnp.float32),
                pltpu.VMEM((2, crows, w), jnp.float32),
                pltpu.VMEM((2, crows, w), jnp.float32),
                pltpu.VMEM((2, crows, w), jnp.float32),
                pltpu.VMEM((2, crows, w), jnp.float32),
                pltpu.VMEM((L,), jnp.float32),
                pltpu.VMEM((ppt * L,), jnp.float32),
                pltpu.VMEM((ppt * L,), jnp.float32),
                pltpu.VMEM((npair * L,), jnp.float32),
                pltpu.VMEM((npair * L,), jnp.float32),
                pltpu.VMEM_SHARED((npair * L,), jnp.float32),
                pltpu.VMEM_SHARED((npair * L,), jnp.float32),
                pltpu.SemaphoreType.DMA((2,)),
                pltpu.SemaphoreType.DMA,
            ],
        )(img, flow, scl)

    def run(img, flow, scale):
        scl = jnp.full((L,), scale, jnp.float32)
        return warp(img, flow, scl)

    return run


_run = _build(8, 3, 256, 256, 8)


def kernel(img, flow, scale):
    return _run(img, flow, scale)


# speedup vs baseline: 1.1921x; 1.0302x over previous
import functools

import jax
import jax.numpy as jnp
from jax import lax
from jax.experimental import pallas as pl
from jax.experimental.pallas import tpu as pltpu
from jax.experimental.pallas import tpu_sc as plsc

SCALE_CONST = 4
NC, NS, L = 2, 16, 16
NW = NC * NS


def _build(batch, chans, h, w, crows, interpret=False):
    n = h * w
    chunk = crows * w
    oh, ow = h * SCALE_CONST, w * SCALE_CONST
    rb = oh // NW
    vpr = w // L
    nchunk = n // chunk
    npair = batch * nchunk
    ppt = npair // NS
    assert n % chunk == 0 and oh % NW == 0 and w % L == 0
    assert npair % NS == 0
    jshift = nchunk.bit_length() - 1
    assert (1 << jshift) == nchunk

    def body(img_hbm, flow_hbm, scl_hbm, out_hbm,
             acc, fyb, fxb, cbufs, sclb, minb, maxb, mint, maxt,
             shmin, shmax, isem, osem):
        cid = lax.axis_index("c")
        sid = lax.axis_index("s")
        wid = sid * NC + cid
        lo = wid * rb

        def any_true(mvec):
            return plsc.all_reduce_population_count(mvec)[0] > 0

        pltpu.sync_copy(scl_hbm, sclb)
        scale_v = sclb[...]

        iota_f = lax.iota(jnp.int32, L).astype(jnp.float32)
        lo_f = lo.astype(jnp.float32)
        band_lo = jnp.full((L,), lo_f - 1.0, jnp.float32)
        band_hi = jnp.full((L,), lo_f + rb, jnp.float32)
        zeros = jnp.zeros((L,), jnp.float32)
        scope = jax.named_scope

        def fy_src(b, j):
            return flow_hbm.at[b, 1, pl.ds(j * crows, crows), :]

        def p0_src(i):
            k = sid * ppt + i
            b = k >> jshift
            return fy_src(b, k - (b << jshift))

        def p0_minmax(i, par):
            k = sid * ppt + i
            b = k >> jshift
            jj = k - (b << jshift)
            pltpu.make_async_copy(p0_src(i), fyb.at[par], isem.at[par]).wait()

            def mm_row(r, carry):
                row_f = (jj * crows + r).astype(jnp.float32)
                gy = jnp.full((L,), row_f, jnp.float32)

                def mm(v, rc):
                    mn, mx = rc
                    y = gy + fyb[par, r, pl.ds(v * L, L)]
                    return jnp.minimum(mn, y), jnp.maximum(mx, y)
                return plsc.parallel_loop(
                    0, vpr, unroll=2, carry=carry)(mm)
            mn, mx = lax.fori_loop(
                0, crows, mm_row,
                (jnp.full((L,), 3.0e38, jnp.float32),
                 jnp.full((L,), -3.0e38, jnp.float32)))
            minb[pl.ds(i * L, L)] = mn * scale_v
            maxb[pl.ds(i * L, L)] = mx * scale_v

        with scope("phase0_range"):
            pltpu.async_copy(p0_src(0), fyb.at[0], isem.at[0])
            for i in range(ppt):
                if i + 1 < ppt:
                    pltpu.async_copy(p0_src(i + 1), fyb.at[(i + 1) & 1],
                                     isem.at[(i + 1) & 1])
                p0_minmax(i, i & 1)
            pltpu.sync_copy(minb, shmin.at[pl.ds(sid * ppt * L, ppt * L)])
            pltpu.sync_copy(maxb, shmax.at[pl.ds(sid * ppt * L, ppt * L)])
            plsc.subcore_barrier()
            pltpu.sync_copy(shmin, mint)
            pltpu.sync_copy(shmax, maxt)

        def zero_rows(r0, nrows):
            def zloop(i, c):
                for k in range(ow // (8 * L)):
                    off = k * (8 * L)
                    for kk in range(8):
                        acc[r0 + i, pl.ds(off + kk * L, L)] = zeros
                return c
            lax.fori_loop(0, nrows, zloop, 0)

        with scope("zero_init"):
            zero_rows(0, chans * rb)

        def chunk_srcs(b, j):
            rsl = pl.ds(j * crows, crows)
            return ([flow_hbm.at[b, 1, rsl, :], flow_hbm.at[b, 0, rsl, :]]
                    + [img_hbm.at[b, ch, rsl, :] for ch in range(chans)])

        def fire(b, j, par):
            for src, dst in zip(chunk_srcs(b, j), [fyb, fxb] + cbufs):
                pltpu.async_copy(src, dst.at[par], isem.at[par])

        def drain(b, j, par):
            for src, dst in zip(chunk_srcs(b, j), [fyb, fxb] + cbufs):
                pltpu.make_async_copy(src, dst.at[par], isem.at[par]).wait()

        def do_proc(b, j, par):
            with scope("fetch_wait"):
                drain(b, j, par)

            def row_loop(r, cr):
                row_f = (j * crows + r).astype(jnp.float32)
                gy = jnp.full((L,), row_f, jnp.float32)

                def proc(v):
                    sl = pl.ds(v * L, L)
                    y = (gy + fyb[par, r, sl]) * scale_v
                    gx = iota_f + (v * L).astype(jnp.float32)
                    x = (gx + fxb[par, r, sl]) * scale_v
                    xt = x.astype(jnp.int32)
                    x0 = xt - jnp.where(xt.astype(jnp.float32) > x, 1, 0)
                    yt = y.astype(jnp.int32)
                    y0 = yt - jnp.where(yt.astype(jnp.float32) > y, 1, 0)
                    wx1 = x - x0.astype(jnp.float32)
                    wx0 = 1.0 - wx1
                    wy1 = y - y0.astype(jnp.float32)
                    wy0 = 1.0 - wy1
                    vals = [cb[par, r, sl] for cb in cbufs]
                    ly = y0 - lo
                    x1 = x0 + 1
                    mx0 = (x0 >= 0) & (x0 < ow)
                    mx1 = (x1 >= 0) & (x1 < ow)
                    cx0 = jnp.where(mx0, x0, 0)
                    cx1 = jnp.where(mx1, x1, 0)
                    for lyv, wy in ((ly, wy0), (ly + 1, wy1)):
                        my = (lyv >= 0) & (lyv < rb)
                        rowi = jnp.where(my, lyv, 0)
                        for coli, mx, wx in ((cx0, mx0, wx0),
                                             (cx1, mx1, wx1)):
                            m = my & mx
                            wgt = wy * wx
                            for ch in range(chans):
                                plsc.addupdate_scatter(
                                    acc, [rowi + ch * rb, coli],
                                    vals[ch] * wgt, mask=m)
                plsc.parallel_loop(0, vpr, unroll=2)(proc)
                return cr
            with scope("proc"):
                lax.fori_loop(0, crows, row_loop, 0)

        def flush_srcs(b):
            return [(acc.at[pl.ds(ch * rb, rb), :],
                     out_hbm.at[b, ch, pl.ds(lo, rb), :])
                    for ch in range(chans)]

        def per_batch(b, carry):
            @pl.when(b > 0)
            def _drain_flush():
                with scope("flush_zero"):
                    for ch, (src, dst) in enumerate(flush_srcs(b - 1)):
                        pltpu.make_async_copy(src, dst, osem).wait()
                        zero_rows(ch * rb, rb)

            def chunk_loop(j, c):
                pend, buf = c
                k16 = (b * nchunk + j) * L
                cmin = mint[pl.ds(k16, L)]
                cmax = maxt[pl.ds(k16, L)]
                hit = any_true((cmax >= band_lo) & (cmin < band_hi))

                @pl.when(hit)
                def _advance():
                    fire(b, j, buf)

                    @pl.when(pend >= 0)
                    def _proc_pending():
                        do_proc(b, pend, 1 - buf)
                return (jnp.where(hit, j, pend),
                        jnp.where(hit, 1 - buf, buf))
            pend, buf = lax.fori_loop(
                0, nchunk, chunk_loop,
                (jnp.int32(-1), jnp.int32(0)))

            @pl.when(pend >= 0)
            def _proc_tail():
                do_proc(b, pend, 1 - buf)

            for src, dst in flush_srcs(b):
                pltpu.async_copy(src, dst, osem)
            return carry

        lax.fori_loop(0, batch, per_batch, 0)

        with scope("final_drain"):
            for src, dst in flush_srcs(batch - 1):
                pltpu.make_async_copy(src, dst, osem).wait()

    mesh = plsc.VectorSubcoreMesh(core_axis_name="c", subcore_axis_name="s",
                                  num_cores=NC, num_subcores=NS)

    def wrapped(img_hbm, flow_hbm, scl_hbm, out_hbm,
                acc, fyb, fxb, c0, c1, c2, sclb,
                minb, maxb, mint, maxt, shmin, shmax, isem, osem):
        return body(img_hbm, flow_hbm, scl_hbm, out_hbm,
                    acc, fyb, fxb, [c0, c1, c2], sclb,
                    minb, maxb, mint, maxt, shmin, shmax, isem, osem)

    @jax.jit
    def warp(img, flow, scl):
        return pl.kernel(
            wrapped,
            out_type=jax.ShapeDtypeStruct((batch, chans, oh, ow),
                                          jnp.float32),
            mesh=mesh,
            interpret=interpret,
            compiler_params=pltpu.CompilerParams(needs_layout_passes=False),
            scratch_types=[
                pltpu.VMEM((chans * rb, ow), jnp.float32),
                pltpu.VMEM((2, crows, w), jnp.float32),
                pltpu.VMEM((2, crows, w), jnp.float32),
                pltpu.VMEM((2, crows, w), jnp.float32),
                pltpu.VMEM((2, crows, w), jnp.float32),
                pltpu.VMEM((2, crows, w), jnp.float32),
                pltpu.VMEM((L,), jnp.float32),
                pltpu.VMEM((ppt * L,), jnp.float32),
                pltpu.VMEM((ppt * L,), jnp.float32),
                pltpu.VMEM((npair * L,), jnp.float32),
                pltpu.VMEM((npair * L,), jnp.float32),
                pltpu.VMEM_SHARED((npair * L,), jnp.float32),
                pltpu.VMEM_SHARED((npair * L,), jnp.float32),
                pltpu.SemaphoreType.DMA((2,)),
                pltpu.SemaphoreType.DMA,
            ],
        )(img, flow, scl)

    def run(img, flow, scale):
        scl = jnp.full((L,), scale, jnp.float32)
        return warp(img, flow, scl)

    return run


_run = _build(8, 3, 256, 256, 8)


def kernel(img, flow, scale):
    return _run(img, flow, scale)
